# baseline (speedup 1.0000x reference)
import functools

import jax
import jax.numpy as jnp
from jax import lax
from jax.experimental import pallas as pl
from jax.experimental.pallas import tpu as pltpu
from jax.experimental.pallas import tpu_sc as plsc

N_ROWS = 8192
C_DIM = 256
K_CODES = 8192
ROW_TILE = 256
K_CHUNK = 1024
N_TILES = N_ROWS // ROW_TILE
N_CHUNKS = K_CODES // K_CHUNK


def _argmin_body(x_ref, cb_ref, idx_ref, mind2_ref):
    xt = x_ref[...]
    x_sq = jnp.sum(xt * xt, axis=1, keepdims=True)

    def chunk(c, carry):
        best_d, best_i = carry
        cb = cb_ref[pl.ds(c * K_CHUNK, K_CHUNK), :]
        cb_sq = jnp.sum(cb * cb, axis=1)
        cross = lax.dot_general(
            xt, cb, (((1,), (1,)), ((), ())),
            preferred_element_type=jnp.float32)
        d2 = x_sq - 2.0 * cross + cb_sq[None, :]
        dist = jnp.sqrt(jnp.maximum(d2, 0.0))
        loc_min = jnp.min(dist, axis=1)
        kidx = lax.broadcasted_iota(jnp.int32, dist.shape, 1)
        loc_arg = jnp.min(
            jnp.where(dist == loc_min[:, None], kidx, K_CODES), axis=1)
        loc_arg = loc_arg + c * K_CHUNK
        take = loc_min < best_d
        return (jnp.where(take, loc_min, best_d),
                jnp.where(take, loc_arg, best_i))

    init = (jnp.full((ROW_TILE,), jnp.inf, jnp.float32),
            jnp.zeros((ROW_TILE,), jnp.int32))
    best_d, best_i = lax.fori_loop(0, N_CHUNKS, chunk, init)
    idx_ref[0, 0, :] = best_i
    mind2_ref[0, 0, :] = best_d * best_d


def _argmin_call(xp, codebook):
    return pl.pallas_call(
        _argmin_body,
        grid=(N_TILES,),
        in_specs=[
            pl.BlockSpec((ROW_TILE, C_DIM), lambda i: (i, 0)),
            pl.BlockSpec((K_CODES, C_DIM), lambda i: (0, 0)),
        ],
        out_specs=[
            pl.BlockSpec((1, 1, ROW_TILE), lambda i: (i, 0, 0)),
            pl.BlockSpec((1, 1, ROW_TILE), lambda i: (i, 0, 0)),
        ],
        out_shape=[
            jax.ShapeDtypeStruct((N_TILES, 1, ROW_TILE), jnp.int32),
            jax.ShapeDtypeStruct((N_TILES, 1, ROW_TILE), jnp.float32),
        ],
    )(xp, codebook)


def _make_sc_gather():
    info = plsc.get_sparse_core_info()
    nw = info.num_cores * info.num_subcores
    b_per_w = N_ROWS // nw
    mesh = plsc.VectorSubcoreMesh(core_axis_name="c", subcore_axis_name="s")

    @functools.partial(
        pl.kernel, mesh=mesh,
        out_type=jax.ShapeDtypeStruct((N_ROWS, C_DIM), jnp.float32),
        scratch_types=[
            pltpu.VMEM((b_per_w,), jnp.int32),
            pltpu.VMEM((b_per_w, C_DIM), jnp.float32),
            pltpu.SemaphoreType.DMA,
        ],
    )
    def gather(table_hbm, idx_hbm, out_hbm, idx_v, rows_v, sem):
        wid = lax.axis_index("s") * info.num_cores + lax.axis_index("c")
        base = wid * b_per_w
        pltpu.sync_copy(idx_hbm.at[pl.ds(base, b_per_w)], idx_v)
        pltpu.async_copy(table_hbm.at[idx_v], rows_v, sem).wait()
        pltpu.sync_copy(rows_v, out_hbm.at[pl.ds(base, b_per_w)])

    return gather


_sc_gather = None


def kernel(x, codebook):
    global _sc_gather
    if _sc_gather is None:
        _sc_gather = _make_sc_gather()
    B, C, L = x.shape
    xp = jnp.transpose(x, (0, 2, 1)).reshape(N_ROWS, C_DIM)
    idx3, mind2 = _argmin_call(xp, codebook)
    idx_flat = idx3.reshape(N_ROWS)
    quant = _sc_gather(codebook, idx_flat)
    loss = jnp.sum(mind2) / (N_ROWS * C_DIM)
    quant_st = jnp.transpose(quant.reshape(B, L, C), (0, 2, 1))
    return quant_st, loss, loss, idx_flat.reshape(B, L)
